# baseline (speedup 1.0000x reference)
import functools

import jax
import jax.numpy as jnp
from jax import lax
from jax.experimental import pallas as pl
from jax.experimental.pallas import tpu as pltpu
from jax.experimental.pallas import tpu_sc as plsc

_D = 2304
_S = 4
_T = 2048
_NW = 32
_BPW = _S * _T // _NW
_CHUNK = 16
_NCHUNK = _BPW // _CHUNK
_NBUF = 3


def _make_gather():
  mesh = plsc.VectorSubcoreMesh(core_axis_name="c", subcore_axis_name="s")

  @functools.partial(
      pl.kernel,
      mesh=mesh,
      out_type=jax.ShapeDtypeStruct((_S, _T, _D), jnp.float32),
      scratch_types=[
          pltpu.VMEM((_BPW,), jnp.int32),
          pltpu.VMEM((_NBUF, _CHUNK, _D), jnp.float32),
      ] + [pltpu.SemaphoreType.DMA] * (2 * _NBUF),
  )
  def gather_kernel(idx_hbm, table_hbm, out_hbm, idx_v, rows_v, *sems):
    gsems = sems[:_NBUF]
    ssems = sems[_NBUF:]
    wid = lax.axis_index("s") * 2 + lax.axis_index("c")
    base = wid * _BPW
    seq = base // _T
    off = base % _T
    pltpu.sync_copy(idx_hbm.at[seq, pl.ds(off, _BPW)], idx_v)

    def fire_gather(c):
      b = c % _NBUF
      return pltpu.async_copy(
          table_hbm.at[idx_v.at[pl.ds(c * _CHUNK, _CHUNK)]],
          rows_v.at[b], gsems[b])

    def fire_scatter(c):
      b = c % _NBUF
      return pltpu.async_copy(
          rows_v.at[b],
          out_hbm.at[seq, pl.ds(off + c * _CHUNK, _CHUNK)], ssems[b])

    gathers = [None] * _NBUF
    scatters = [None] * _NBUF
    for t in range(_NCHUNK + _NBUF - 1):
      if t < _NCHUNK:
        b = t % _NBUF
        if scatters[b] is not None:
          scatters[b].wait()
          scatters[b] = None
        gathers[b] = fire_gather(t)
      d = t - (_NBUF - 1)
      if d >= 0:
        bd = d % _NBUF
        gathers[bd].wait()
        scatters[bd] = fire_scatter(d)
    for s in scatters:
      if s is not None:
        s.wait()

  return gather_kernel


_gather = _make_gather()


def kernel(token_ids, table):
  return _gather(token_ids.astype(jnp.int32), table)


# speedup vs baseline: 1.0113x; 1.0113x over previous
import jax
import jax.numpy as jnp
from jax import lax
from jax.experimental import pallas as pl
from jax.experimental.pallas import tpu as pltpu
from jax.experimental.pallas import tpu_sc as plsc
from jax._src.pallas import mpmd as plmpmd

_D = 2304
_B = 8192
_NW = 32

_R_SCS = 1536
_R_TEC = _B - _R_SCS

_BPW = _R_TEC // _NW
_CHUNK = 16
_NCHUNK = _BPW // _CHUNK
_NBUF = 2

_SPC = _R_SCS // 2
_SCHUNK = 128
_SNCHUNK = _SPC // _SCHUNK


def _make_gather():
  vmesh = plsc.VectorSubcoreMesh(core_axis_name="c", subcore_axis_name="s")
  smesh = plsc.ScalarSubcoreMesh(axis_name="c", num_cores=2)

  tec_vmem = pltpu.MemorySpace.VMEM @ vmesh
  scs_smem = pltpu.MemorySpace.SMEM @ smesh
  scratch = [
      tec_vmem((_BPW,), jnp.int32),
      tec_vmem((_NBUF, _CHUNK, _D), jnp.float32),
  ] + [pltpu.SemaphoreType.DMA @ vmesh] * (2 * _NBUF) + [
      scs_smem((_SPC,), jnp.int32),
      pltpu.MemorySpace.VMEM_SHARED((2, _SCHUNK, _D), jnp.float32),
      pltpu.SemaphoreType.DMA @ smesh,
      pltpu.SemaphoreType.DMA @ smesh,
      pltpu.SemaphoreType.DMA @ smesh,
  ]

  def tec_fn(idx_hbm, table_hbm, out_hbm, idx_v, rows_v,
             g0, g1, s0, s1, *_scs_scratch):
    gsems = (g0, g1)
    ssems = (s0, s1)
    wid = lax.axis_index("s") * 2 + lax.axis_index("c")
    base = wid * _BPW
    pltpu.sync_copy(idx_hbm.at[pl.ds(base, _BPW)], idx_v)

    def fire_gather(c):
      b = c % _NBUF
      return pltpu.async_copy(
          table_hbm.at[idx_v.at[pl.ds(c * _CHUNK, _CHUNK)]],
          rows_v.at[b], gsems[b])

    def fire_scatter(c):
      b = c % _NBUF
      return pltpu.async_copy(
          rows_v.at[b],
          out_hbm.at[pl.ds(base + c * _CHUNK, _CHUNK)], ssems[b])

    gathers = [None] * _NBUF
    scatters = [None] * _NBUF
    for t in range(_NCHUNK + _NBUF - 1):
      if t < _NCHUNK:
        b = t % _NBUF
        if scatters[b] is not None:
          scatters[b].wait()
          scatters[b] = None
        gathers[b] = fire_gather(t)
      d = t - (_NBUF - 1)
      if d >= 0:
        bd = d % _NBUF
        gathers[bd].wait()
        scatters[bd] = fire_scatter(d)
    for s in scatters:
      if s is not None:
        s.wait()

  def scs_fn(idx_hbm, table_hbm, out_hbm, _idx_v, _rows_v,
             _g0, _g1, _s0, _s1,
             idx_s, rows_sh, gsem, ssem0, ssem1):
    cid = lax.axis_index("c")
    base = _R_TEC + cid * _SPC
    pltpu.sync_copy(idx_hbm.at[pl.ds(base, _SPC)], idx_s)
    ssems = (ssem0, ssem1)

    scatters = [None, None]
    for c in range(_SNCHUNK):
      b = c % 2
      if scatters[b] is not None:
        scatters[b].wait()
        scatters[b] = None

      def issue_row(j, carry, c=c, b=b):
        row = idx_s[c * _SCHUNK + j]
        pltpu.async_copy(
            table_hbm.at[pl.ds(row, 1)],
            rows_sh.at[b].at[pl.ds(j, 1)], gsem)
        return carry

      lax.fori_loop(0, _SCHUNK, issue_row, 0)
      pltpu.make_async_copy(
          table_hbm.at[pl.ds(0, _SCHUNK)], rows_sh.at[b], gsem).wait()
      scatters[b] = pltpu.async_copy(
          rows_sh.at[b],
          out_hbm.at[pl.ds(base + c * _SCHUNK, _SCHUNK)], ssems[b])
    for s in scatters:
      if s is not None:
        s.wait()

  return plmpmd.mpmd_map(
      [(smesh, scs_fn), (vmesh, tec_fn)],
      out_types=jax.ShapeDtypeStruct((_B, _D), jnp.float32),
      scratch_types=scratch,
  )


_gather = _make_gather()


def kernel(token_ids, table):
  flat_ids = token_ids.reshape(-1).astype(jnp.int32)
  out = _gather(flat_ids, table)
  return out.reshape(token_ids.shape + (table.shape[1],))


# speedup vs baseline: 1.0120x; 1.0007x over previous
import jax
import jax.numpy as jnp
from jax import lax
from jax.experimental import pallas as pl
from jax.experimental.pallas import tpu as pltpu
from jax.experimental.pallas import tpu_sc as plsc
from jax._src.pallas import mpmd as plmpmd

_D = 2304
_B = 8192
_NW = 32

_R_SCS = 1024
_R_TEC = _B - _R_SCS

_BPW = _R_TEC // _NW
_CHUNK = 16
_NCHUNK = _BPW // _CHUNK
_NBUF = 2

_SPC = _R_SCS // 2
_SCHUNK = 128
_SNCHUNK = _SPC // _SCHUNK


def _make_gather():
  vmesh = plsc.VectorSubcoreMesh(core_axis_name="c", subcore_axis_name="s")
  smesh = plsc.ScalarSubcoreMesh(axis_name="c", num_cores=2)

  tec_vmem = pltpu.MemorySpace.VMEM @ vmesh
  scs_smem = pltpu.MemorySpace.SMEM @ smesh
  scratch = [
      tec_vmem((_BPW,), jnp.int32),
      tec_vmem((_NBUF, _CHUNK, _D), jnp.float32),
  ] + [pltpu.SemaphoreType.DMA @ vmesh] * (2 * _NBUF) + [
      scs_smem((_SPC,), jnp.int32),
      pltpu.MemorySpace.VMEM_SHARED((2, _SCHUNK, _D), jnp.float32),
      pltpu.SemaphoreType.DMA @ smesh,
      pltpu.SemaphoreType.DMA @ smesh,
      pltpu.SemaphoreType.DMA @ smesh,
  ]

  def tec_fn(idx_hbm, table_hbm, out_hbm, idx_v, rows_v,
             g0, g1, s0, s1, *_scs_scratch):
    gsems = (g0, g1)
    ssems = (s0, s1)
    wid = lax.axis_index("s") * 2 + lax.axis_index("c")
    base = wid * _BPW
    pltpu.sync_copy(idx_hbm.at[pl.ds(base, _BPW)], idx_v)

    def fire_gather(c):
      b = c % _NBUF
      return pltpu.async_copy(
          table_hbm.at[idx_v.at[pl.ds(c * _CHUNK, _CHUNK)]],
          rows_v.at[b], gsems[b])

    def fire_scatter(c):
      b = c % _NBUF
      return pltpu.async_copy(
          rows_v.at[b],
          out_hbm.at[pl.ds(base + c * _CHUNK, _CHUNK)], ssems[b])

    gathers = [None] * _NBUF
    scatters = [None] * _NBUF
    for t in range(_NCHUNK + _NBUF - 1):
      if t < _NCHUNK:
        b = t % _NBUF
        if scatters[b] is not None:
          scatters[b].wait()
          scatters[b] = None
        gathers[b] = fire_gather(t)
      d = t - (_NBUF - 1)
      if d >= 0:
        bd = d % _NBUF
        gathers[bd].wait()
        scatters[bd] = fire_scatter(d)
    for s in scatters:
      if s is not None:
        s.wait()

  def scs_fn(idx_hbm, table_hbm, out_hbm, _idx_v, _rows_v,
             _g0, _g1, _s0, _s1,
             idx_s, rows_sh, gsem, ssem0, ssem1):
    cid = lax.axis_index("c")
    base = _R_TEC + cid * _SPC
    pltpu.sync_copy(idx_hbm.at[pl.ds(base, _SPC)], idx_s)
    ssems = (ssem0, ssem1)

    scatters = [None, None]
    for c in range(_SNCHUNK):
      b = c % 2
      if scatters[b] is not None:
        scatters[b].wait()
        scatters[b] = None

      def issue_row(j, carry, c=c, b=b):
        row = idx_s[c * _SCHUNK + j]
        pltpu.async_copy(
            table_hbm.at[pl.ds(row, 1)],
            rows_sh.at[b].at[pl.ds(j, 1)], gsem)
        return carry

      lax.fori_loop(0, _SCHUNK, issue_row, 0)
      pltpu.make_async_copy(
          table_hbm.at[pl.ds(0, _SCHUNK)], rows_sh.at[b], gsem).wait()
      scatters[b] = pltpu.async_copy(
          rows_sh.at[b],
          out_hbm.at[pl.ds(base + c * _SCHUNK, _SCHUNK)], ssems[b])
    for s in scatters:
      if s is not None:
        s.wait()

  return plmpmd.mpmd_map(
      [(smesh, scs_fn), (vmesh, tec_fn)],
      out_types=jax.ShapeDtypeStruct((_B, _D), jnp.float32),
      scratch_types=scratch,
  )


_gather = _make_gather()


def kernel(token_ids, table):
  flat_ids = token_ids.reshape(-1).astype(jnp.int32)
  out = _gather(flat_ids, table)
  return out.reshape(token_ids.shape + (table.shape[1],))


# speedup vs baseline: 1.0190x; 1.0069x over previous
import jax
import jax.numpy as jnp
from jax import lax
from jax.experimental import pallas as pl
from jax.experimental.pallas import tpu as pltpu
from jax.experimental.pallas import tpu_sc as plsc
from jax._src.pallas import mpmd as plmpmd

_D = 2304
_B = 8192
_NW = 32

_R_SCS = 512
_R_TEC = _B - _R_SCS

_BPW = _R_TEC // _NW
_CHUNK = 16
_NCHUNK = _BPW // _CHUNK
_NBUF = 3

_SPC = _R_SCS // 2
_SCHUNK = 64
_SNCHUNK = _SPC // _SCHUNK


def _make_gather():
  vmesh = plsc.VectorSubcoreMesh(core_axis_name="c", subcore_axis_name="s")
  smesh = plsc.ScalarSubcoreMesh(axis_name="c", num_cores=2)

  tec_vmem = pltpu.MemorySpace.VMEM @ vmesh
  scs_smem = pltpu.MemorySpace.SMEM @ smesh
  scratch = [
      tec_vmem((_BPW,), jnp.int32),
      tec_vmem((_NBUF, _CHUNK, _D), jnp.float32),
  ] + [pltpu.SemaphoreType.DMA @ vmesh] * (2 * _NBUF) + [
      scs_smem((_SPC,), jnp.int32),
      pltpu.MemorySpace.VMEM_SHARED((2, _SCHUNK, _D), jnp.float32),
      pltpu.SemaphoreType.DMA @ smesh,
      pltpu.SemaphoreType.DMA @ smesh,
      pltpu.SemaphoreType.DMA @ smesh,
  ]

  def tec_fn(idx_hbm, table_hbm, out_hbm, idx_v, rows_v,
             g0, g1, g2, s0, s1, s2, *_scs_scratch):
    gsems = (g0, g1, g2)
    ssems = (s0, s1, s2)
    wid = lax.axis_index("s") * 2 + lax.axis_index("c")
    base = wid * _BPW
    pltpu.sync_copy(idx_hbm.at[pl.ds(base, _BPW)], idx_v)

    def fire_gather(c):
      b = c % _NBUF
      return pltpu.async_copy(
          table_hbm.at[idx_v.at[pl.ds(c * _CHUNK, _CHUNK)]],
          rows_v.at[b], gsems[b])

    def fire_scatter(c):
      b = c % _NBUF
      return pltpu.async_copy(
          rows_v.at[b],
          out_hbm.at[pl.ds(base + c * _CHUNK, _CHUNK)], ssems[b])

    gathers = [None] * _NBUF
    scatters = [None] * _NBUF
    for t in range(_NCHUNK + _NBUF - 1):
      if t < _NCHUNK:
        b = t % _NBUF
        if scatters[b] is not None:
          scatters[b].wait()
          scatters[b] = None
        gathers[b] = fire_gather(t)
      d = t - (_NBUF - 1)
      if d >= 0:
        bd = d % _NBUF
        gathers[bd].wait()
        scatters[bd] = fire_scatter(d)
    for s in scatters:
      if s is not None:
        s.wait()

  def scs_fn(idx_hbm, table_hbm, out_hbm, _idx_v, _rows_v,
             _g0, _g1, _g2, _s0, _s1, _s2,
             idx_s, rows_sh, gsem, ssem0, ssem1):
    cid = lax.axis_index("c")
    base = _R_TEC + cid * _SPC
    pltpu.sync_copy(idx_hbm.at[pl.ds(base, _SPC)], idx_s)
    ssems = (ssem0, ssem1)

    scatters = [None, None]
    for c in range(_SNCHUNK):
      b = c % 2
      if scatters[b] is not None:
        scatters[b].wait()
        scatters[b] = None

      def issue_row(j, carry, c=c, b=b):
        row = idx_s[c * _SCHUNK + j]
        pltpu.async_copy(
            table_hbm.at[pl.ds(row, 1)],
            rows_sh.at[b].at[pl.ds(j, 1)], gsem)
        return carry

      lax.fori_loop(0, _SCHUNK, issue_row, 0)
      pltpu.make_async_copy(
          table_hbm.at[pl.ds(0, _SCHUNK)], rows_sh.at[b], gsem).wait()
      scatters[b] = pltpu.async_copy(
          rows_sh.at[b],
          out_hbm.at[pl.ds(base + c * _SCHUNK, _SCHUNK)], ssems[b])
    for s in scatters:
      if s is not None:
        s.wait()

  return plmpmd.mpmd_map(
      [(smesh, scs_fn), (vmesh, tec_fn)],
      out_types=jax.ShapeDtypeStruct((_B, _D), jnp.float32),
      scratch_types=scratch,
  )


_gather = _make_gather()


def kernel(token_ids, table):
  flat_ids = token_ids.reshape(-1).astype(jnp.int32)
  out = _gather(flat_ids, table)
  return out.reshape(token_ids.shape + (table.shape[1],))
